# Initial kernel scaffold; baseline (speedup 1.0000x reference)
import jax
import jax.numpy as jnp
from jax.experimental import pallas as pl


def kernel(points, theta, basis):
    raise NotImplementedError("write your pallas kernel here")


# speedup vs baseline: 258.7704x; 258.7704x over previous
import functools

import jax
import jax.numpy as jnp
from jax import lax
from jax.experimental import pallas as pl
from jax.experimental.pallas import tpu as pltpu
from jax.experimental.pallas import tpu_sc as plsc

NDIM = 1
NC = 64
NSTEP = 8
NTHETA = 64
NPOINTS = 100000

CHUNK = 20000
NCHUNK = NPOINTS // CHUNK
NWORKERS = 32
T_PER_W = NTHETA // NWORKERS


def _expm_taylor_2x2(A):
    n = A.shape[-1]
    I = jnp.broadcast_to(jnp.eye(n, dtype=A.dtype), A.shape)
    out = I
    term = I
    for k in range(1, 13):
        term = jnp.matmul(term, A) / k
        out = out + term
    return out


def _build_tables(theta, basis):
    n_theta = theta.shape[0]
    Avees = jnp.matmul(basis, theta.T)
    As = Avees.T.reshape(n_theta * NC, 1, NDIM + 1)
    zero_row = jnp.zeros((n_theta * NC, 1, NDIM + 1), dtype=theta.dtype)
    AsSquare = jnp.concatenate([As, zero_row], axis=1)
    Trels = _expm_taylor_2x2(AsSquare * (1.0 / NSTEP))
    return Trels[:, 0, 0], Trels[:, 0, 1]


def _sc_body(points_hbm, ctab_hbm, dtab_hbm, out_hbm,
             ctab_v, dtab_v, pts_v, obuf0, obuf1):
    wid = lax.axis_index("s") * 2 + lax.axis_index("c")
    t0 = wid * T_PER_W
    pltpu.sync_copy(ctab_hbm.at[pl.ds(t0 * NC, T_PER_W * NC)], ctab_v)
    pltpu.sync_copy(dtab_hbm.at[pl.ds(t0 * NC, T_PER_W * NC)], dtab_v)
    obufs = (obuf0, obuf1)
    for chunk in range(NCHUNK):
        pltpu.sync_copy(points_hbm.at[pl.ds(chunk * CHUNK, CHUNK)], pts_v)
        for tt in range(T_PER_W):
            obuf = obufs[tt]

            def body(i, _, tt=tt, obuf=obuf):
                x = pts_v[pl.ds(i * 16, 16)]
                for _step in range(NSTEP):
                    zi = (x * float(NC)).astype(jnp.int32)
                    idx = jnp.clip(zi, 0, NC - 1) + (tt * NC)
                    cv = plsc.load_gather(ctab_v, [idx])
                    dv = plsc.load_gather(dtab_v, [idx])
                    x = cv * x + dv
                obuf[pl.ds(i * 16, 16)] = x
                return 0

            lax.fori_loop(0, CHUNK // 16, body, 0)
        for tt in range(T_PER_W):
            pltpu.sync_copy(
                obufs[tt],
                out_hbm.at[pl.ds((t0 + tt) * NPOINTS + chunk * CHUNK, CHUNK)])


@functools.partial(
    pl.kernel,
    out_type=jax.ShapeDtypeStruct((NTHETA * NPOINTS,), jnp.float32),
    mesh=plsc.VectorSubcoreMesh(core_axis_name="c", subcore_axis_name="s"),
    compiler_params=pltpu.CompilerParams(needs_layout_passes=False),
    scratch_types=[
        pltpu.VMEM((T_PER_W * NC,), jnp.float32),
        pltpu.VMEM((T_PER_W * NC,), jnp.float32),
        pltpu.VMEM((CHUNK,), jnp.float32),
        pltpu.VMEM((CHUNK,), jnp.float32),
        pltpu.VMEM((CHUNK,), jnp.float32),
    ],
)
def _sc_transform(points_hbm, ctab_hbm, dtab_hbm, out_hbm,
                  ctab_v, dtab_v, pts_v, obuf0, obuf1):
    _sc_body(points_hbm, ctab_hbm, dtab_hbm, out_hbm,
             ctab_v, dtab_v, pts_v, obuf0, obuf1)


def kernel(points, theta, basis):
    ctab, dtab = _build_tables(theta, basis)
    out = _sc_transform(points.reshape(NPOINTS), ctab, dtab)
    return out.reshape(NTHETA, NDIM, NPOINTS)


# speedup vs baseline: 918.8942x; 3.5510x over previous
import functools

import jax
import jax.numpy as jnp
from jax import lax
from jax.experimental import pallas as pl
from jax.experimental.pallas import tpu as pltpu
from jax.experimental.pallas import tpu_sc as plsc

NDIM = 1
NC = 64
NSTEP = 8
NTHETA = 64
NPOINTS = 100000

CHUNK = 20000
NCHUNK = NPOINTS // CHUNK
NWORKERS = 32
T_PER_W = NTHETA // NWORKERS
UNROLL = 2


def _expm_taylor_2x2(A):
    n = A.shape[-1]
    I = jnp.broadcast_to(jnp.eye(n, dtype=A.dtype), A.shape)
    out = I
    term = I
    for k in range(1, 13):
        term = jnp.matmul(term, A) / k
        out = out + term
    return out


def _build_tables(theta, basis):
    n_theta = theta.shape[0]
    Avees = jnp.matmul(basis, theta.T)
    As = Avees.T.reshape(n_theta * NC, 1, NDIM + 1)
    zero_row = jnp.zeros((n_theta * NC, 1, NDIM + 1), dtype=theta.dtype)
    AsSquare = jnp.concatenate([As, zero_row], axis=1)
    Trels = _expm_taylor_2x2(AsSquare * (1.0 / NSTEP))
    return Trels[:, 0, 0], Trels[:, 0, 1]


def _sc_body(points_hbm, ctab_hbm, dtab_hbm, out_hbm,
             c0_v, c1_v, d0_v, d1_v, pts_v, obuf0, obuf1):
    wid = lax.axis_index("s") * 2 + lax.axis_index("c")
    t0 = wid * T_PER_W
    pltpu.sync_copy(ctab_hbm.at[pl.ds(t0 * NC, NC)], c0_v)
    pltpu.sync_copy(ctab_hbm.at[pl.ds((t0 + 1) * NC, NC)], c1_v)
    pltpu.sync_copy(dtab_hbm.at[pl.ds(t0 * NC, NC)], d0_v)
    pltpu.sync_copy(dtab_hbm.at[pl.ds((t0 + 1) * NC, NC)], d1_v)
    inv_nc = 1.0 / NC
    for chunk in range(NCHUNK):
        pltpu.sync_copy(points_hbm.at[pl.ds(chunk * CHUNK, CHUNK)], pts_v)

        @plsc.parallel_loop(0, CHUNK // 16, 1, unroll=UNROLL)
        def _(i):
            base = i * 16
            x = pts_v[pl.ds(base, 16)]
            x0 = x
            x1 = x
            for _step in range(NSTEP):
                i0 = jnp.clip(x0, 0.0, float(NC - 1)).astype(jnp.int32)
                i1 = jnp.clip(x1, 0.0, float(NC - 1)).astype(jnp.int32)
                cv0 = plsc.load_gather(c0_v, [i0])
                dv0 = plsc.load_gather(d0_v, [i0])
                cv1 = plsc.load_gather(c1_v, [i1])
                dv1 = plsc.load_gather(d1_v, [i1])
                x0 = cv0 * x0 + dv0
                x1 = cv1 * x1 + dv1
            obuf0[pl.ds(base, 16)] = x0 * inv_nc
            obuf1[pl.ds(base, 16)] = x1 * inv_nc

        pltpu.sync_copy(
            obuf0, out_hbm.at[pl.ds(t0 * NPOINTS + chunk * CHUNK, CHUNK)])
        pltpu.sync_copy(
            obuf1, out_hbm.at[pl.ds((t0 + 1) * NPOINTS + chunk * CHUNK, CHUNK)])


@functools.partial(
    pl.kernel,
    out_type=jax.ShapeDtypeStruct((NTHETA * NPOINTS,), jnp.float32),
    mesh=plsc.VectorSubcoreMesh(core_axis_name="c", subcore_axis_name="s"),
    compiler_params=pltpu.CompilerParams(needs_layout_passes=False),
    scratch_types=[
        pltpu.VMEM((NC,), jnp.float32),
        pltpu.VMEM((NC,), jnp.float32),
        pltpu.VMEM((NC,), jnp.float32),
        pltpu.VMEM((NC,), jnp.float32),
        pltpu.VMEM((CHUNK,), jnp.float32),
        pltpu.VMEM((CHUNK,), jnp.float32),
        pltpu.VMEM((CHUNK,), jnp.float32),
    ],
)
def _sc_transform(points_hbm, ctab_hbm, dtab_hbm, out_hbm,
                  c0_v, c1_v, d0_v, d1_v, pts_v, obuf0, obuf1):
    _sc_body(points_hbm, ctab_hbm, dtab_hbm, out_hbm,
             c0_v, c1_v, d0_v, d1_v, pts_v, obuf0, obuf1)


def kernel(points, theta, basis):
    ctab, dtab = _build_tables(theta, basis)
    out = _sc_transform(points.reshape(NPOINTS) * float(NC),
                        ctab, dtab * float(NC))
    return out.reshape(NTHETA, NDIM, NPOINTS)


# speedup vs baseline: 918.9867x; 1.0001x over previous
import functools

import jax
import jax.numpy as jnp
from jax import lax
from jax.experimental import pallas as pl
from jax.experimental.pallas import tpu as pltpu
from jax.experimental.pallas import tpu_sc as plsc

NDIM = 1
NC = 64
NSTEP = 8
NTHETA = 64
NPOINTS = 100000

CHUNK = 20000
NCHUNK = NPOINTS // CHUNK
NWORKERS = 32
T_PER_W = NTHETA // NWORKERS
UNROLL = 5


def _expm_taylor_2x2(A):
    n = A.shape[-1]
    I = jnp.broadcast_to(jnp.eye(n, dtype=A.dtype), A.shape)
    out = I
    term = I
    for k in range(1, 13):
        term = jnp.matmul(term, A) / k
        out = out + term
    return out


def _build_tables(theta, basis):
    n_theta = theta.shape[0]
    Avees = jnp.matmul(basis, theta.T)
    As = Avees.T.reshape(n_theta * NC, 1, NDIM + 1)
    zero_row = jnp.zeros((n_theta * NC, 1, NDIM + 1), dtype=theta.dtype)
    AsSquare = jnp.concatenate([As, zero_row], axis=1)
    Trels = _expm_taylor_2x2(AsSquare * (1.0 / NSTEP))
    return Trels[:, 0, 0], Trels[:, 0, 1]


def _sc_body(points_hbm, ctab_hbm, dtab_hbm, out_hbm,
             c0_v, c1_v, d0_v, d1_v, pts_v, obuf0, obuf1):
    wid = lax.axis_index("s") * 2 + lax.axis_index("c")
    t0 = wid * T_PER_W
    pltpu.sync_copy(ctab_hbm.at[pl.ds(t0 * NC, NC)], c0_v)
    pltpu.sync_copy(ctab_hbm.at[pl.ds((t0 + 1) * NC, NC)], c1_v)
    pltpu.sync_copy(dtab_hbm.at[pl.ds(t0 * NC, NC)], d0_v)
    pltpu.sync_copy(dtab_hbm.at[pl.ds((t0 + 1) * NC, NC)], d1_v)
    inv_nc = 1.0 / NC
    for chunk in range(NCHUNK):
        pltpu.sync_copy(points_hbm.at[pl.ds(chunk * CHUNK, CHUNK)], pts_v)

        @plsc.parallel_loop(0, CHUNK // 16, 1, unroll=UNROLL)
        def _(i):
            base = i * 16
            x = pts_v[pl.ds(base, 16)]
            x0 = x
            x1 = x
            for _step in range(NSTEP):
                i0 = jnp.clip(x0, 0.0, float(NC - 1)).astype(jnp.int32)
                i1 = jnp.clip(x1, 0.0, float(NC - 1)).astype(jnp.int32)
                cv0 = plsc.load_gather(c0_v, [i0])
                dv0 = plsc.load_gather(d0_v, [i0])
                cv1 = plsc.load_gather(c1_v, [i1])
                dv1 = plsc.load_gather(d1_v, [i1])
                x0 = cv0 * x0 + dv0
                x1 = cv1 * x1 + dv1
            obuf0[pl.ds(base, 16)] = x0 * inv_nc
            obuf1[pl.ds(base, 16)] = x1 * inv_nc

        pltpu.sync_copy(
            obuf0, out_hbm.at[pl.ds(t0 * NPOINTS + chunk * CHUNK, CHUNK)])
        pltpu.sync_copy(
            obuf1, out_hbm.at[pl.ds((t0 + 1) * NPOINTS + chunk * CHUNK, CHUNK)])


@functools.partial(
    pl.kernel,
    out_type=jax.ShapeDtypeStruct((NTHETA * NPOINTS,), jnp.float32),
    mesh=plsc.VectorSubcoreMesh(core_axis_name="c", subcore_axis_name="s"),
    compiler_params=pltpu.CompilerParams(needs_layout_passes=False),
    scratch_types=[
        pltpu.VMEM((NC,), jnp.float32),
        pltpu.VMEM((NC,), jnp.float32),
        pltpu.VMEM((NC,), jnp.float32),
        pltpu.VMEM((NC,), jnp.float32),
        pltpu.VMEM((CHUNK,), jnp.float32),
        pltpu.VMEM((CHUNK,), jnp.float32),
        pltpu.VMEM((CHUNK,), jnp.float32),
    ],
)
def _sc_transform(points_hbm, ctab_hbm, dtab_hbm, out_hbm,
                  c0_v, c1_v, d0_v, d1_v, pts_v, obuf0, obuf1):
    _sc_body(points_hbm, ctab_hbm, dtab_hbm, out_hbm,
             c0_v, c1_v, d0_v, d1_v, pts_v, obuf0, obuf1)


def kernel(points, theta, basis):
    ctab, dtab = _build_tables(theta, basis)
    out = _sc_transform(points.reshape(NPOINTS) * float(NC),
                        ctab, dtab * float(NC))
    return out.reshape(NTHETA, NDIM, NPOINTS)


# speedup vs baseline: 943.2774x; 1.0264x over previous
import functools

import jax
import jax.numpy as jnp
from jax import lax
from jax.experimental import pallas as pl
from jax.experimental.pallas import tpu as pltpu
from jax.experimental.pallas import tpu_sc as plsc

NDIM = 1
NC = 64
NSTEP = 8
NTHETA = 64
NPOINTS = 100000

CHUNK = 10000
NCHUNK = NPOINTS // CHUNK
NWORKERS = 32
T_PER_W = NTHETA // NWORKERS
UNROLL = 5


def _expm_taylor_2x2(A):
    n = A.shape[-1]
    I = jnp.broadcast_to(jnp.eye(n, dtype=A.dtype), A.shape)
    out = I
    term = I
    for k in range(1, 13):
        term = jnp.matmul(term, A) / k
        out = out + term
    return out


def _build_tables(theta, basis):
    n_theta = theta.shape[0]
    Avees = jnp.matmul(basis, theta.T)
    As = Avees.T.reshape(n_theta * NC, 1, NDIM + 1)
    zero_row = jnp.zeros((n_theta * NC, 1, NDIM + 1), dtype=theta.dtype)
    AsSquare = jnp.concatenate([As, zero_row], axis=1)
    Trels = _expm_taylor_2x2(AsSquare * (1.0 / NSTEP))
    return Trels[:, 0, 0], Trels[:, 0, 1]


def _sc_body(points_hbm, ctab_hbm, dtab_hbm, out_hbm,
             c0_v, c1_v, d0_v, d1_v, p0_v, p1_v,
             ob00, ob01, ob10, ob11, sin0, sin1, sout0, sout1):
    wid = lax.axis_index("s") * 2 + lax.axis_index("c")
    t0 = wid * T_PER_W
    pltpu.sync_copy(ctab_hbm.at[pl.ds(t0 * NC, NC)], c0_v)
    pltpu.sync_copy(ctab_hbm.at[pl.ds((t0 + 1) * NC, NC)], c1_v)
    pltpu.sync_copy(dtab_hbm.at[pl.ds(t0 * NC, NC)], d0_v)
    pltpu.sync_copy(dtab_hbm.at[pl.ds((t0 + 1) * NC, NC)], d1_v)
    inv_nc = 1.0 / NC
    pts = (p0_v, p1_v)
    obufs = ((ob00, ob01), (ob10, ob11))
    sins = (sin0, sin1)
    souts = (sout0, sout1)

    def in_copy(chunk, p):
        return pltpu.make_async_copy(
            points_hbm.at[pl.ds(chunk * CHUNK, CHUNK)], pts[p], sins[p])

    def out_copy(chunk, p, tt):
        return pltpu.make_async_copy(
            obufs[tt][p],
            out_hbm.at[pl.ds((t0 + tt) * NPOINTS + chunk * CHUNK, CHUNK)],
            souts[p])

    in_copy(0, 0).start()
    in_copy(1, 1).start()

    def chunk_body(j, carry):
        for p in range(2):
            chunk = 2 * j + p
            in_copy(chunk, p).wait()

            @pl.when(j > 0)
            def _():
                out_copy(chunk, p, 0).wait()
                out_copy(chunk, p, 1).wait()

            pv = pts[p]
            ob0 = obufs[0][p]
            ob1 = obufs[1][p]

            @plsc.parallel_loop(0, CHUNK // 16, 1, unroll=UNROLL)
            def _(i):
                base = i * 16
                x = pv[pl.ds(base, 16)]
                x0 = x
                x1 = x
                for _step in range(NSTEP):
                    i0 = jnp.clip(x0, 0.0, float(NC - 1)).astype(jnp.int32)
                    i1 = jnp.clip(x1, 0.0, float(NC - 1)).astype(jnp.int32)
                    cv0 = plsc.load_gather(c0_v, [i0])
                    dv0 = plsc.load_gather(d0_v, [i0])
                    cv1 = plsc.load_gather(c1_v, [i1])
                    dv1 = plsc.load_gather(d1_v, [i1])
                    x0 = cv0 * x0 + dv0
                    x1 = cv1 * x1 + dv1
                ob0[pl.ds(base, 16)] = x0 * inv_nc
                ob1[pl.ds(base, 16)] = x1 * inv_nc

            out_copy(chunk, p, 0).start()
            out_copy(chunk, p, 1).start()

            @pl.when(j < NCHUNK // 2 - 1)
            def _():
                in_copy(chunk + 2, p).start()
        return carry

    lax.fori_loop(0, NCHUNK // 2, chunk_body, 0)
    for p in range(2):
        out_copy(NCHUNK - 2 + p, p, 0).wait()
        out_copy(NCHUNK - 2 + p, p, 1).wait()


@functools.partial(
    pl.kernel,
    out_type=jax.ShapeDtypeStruct((NTHETA * NPOINTS,), jnp.float32),
    mesh=plsc.VectorSubcoreMesh(core_axis_name="c", subcore_axis_name="s"),
    compiler_params=pltpu.CompilerParams(needs_layout_passes=False),
    scratch_types=[
        pltpu.VMEM((NC,), jnp.float32),
        pltpu.VMEM((NC,), jnp.float32),
        pltpu.VMEM((NC,), jnp.float32),
        pltpu.VMEM((NC,), jnp.float32),
        pltpu.VMEM((CHUNK,), jnp.float32),
        pltpu.VMEM((CHUNK,), jnp.float32),
        pltpu.VMEM((CHUNK,), jnp.float32),
        pltpu.VMEM((CHUNK,), jnp.float32),
        pltpu.VMEM((CHUNK,), jnp.float32),
        pltpu.VMEM((CHUNK,), jnp.float32),
        pltpu.SemaphoreType.DMA,
        pltpu.SemaphoreType.DMA,
        pltpu.SemaphoreType.DMA,
        pltpu.SemaphoreType.DMA,
    ],
)
def _sc_transform(points_hbm, ctab_hbm, dtab_hbm, out_hbm,
                  c0_v, c1_v, d0_v, d1_v, p0_v, p1_v,
                  ob00, ob01, ob10, ob11, sin0, sin1, sout0, sout1):
    _sc_body(points_hbm, ctab_hbm, dtab_hbm, out_hbm,
             c0_v, c1_v, d0_v, d1_v, p0_v, p1_v,
             ob00, ob01, ob10, ob11, sin0, sin1, sout0, sout1)


def kernel(points, theta, basis):
    ctab, dtab = _build_tables(theta, basis)
    out = _sc_transform(points.reshape(NPOINTS) * float(NC),
                        ctab, dtab * float(NC))
    return out.reshape(NTHETA, NDIM, NPOINTS)


# speedup vs baseline: 943.6712x; 1.0004x over previous
import functools

import jax
import jax.numpy as jnp
from jax import lax
from jax.experimental import pallas as pl
from jax.experimental.pallas import tpu as pltpu
from jax.experimental.pallas import tpu_sc as plsc

NDIM = 1
NC = 64
NSTEP = 8
NTHETA = 64
NPOINTS = 100000

CHUNK = 10000
NCHUNK = NPOINTS // CHUNK
NWORKERS = 32
T_PER_W = NTHETA // NWORKERS
UNROLL = 5


def _expm_taylor_2x2(A):
    n = A.shape[-1]
    I = jnp.broadcast_to(jnp.eye(n, dtype=A.dtype), A.shape)
    out = I
    term = I
    for k in range(1, 13):
        term = jnp.matmul(term, A) / k
        out = out + term
    return out


def _build_tables(theta, basis):
    n_theta = theta.shape[0]
    Avees = jnp.matmul(basis, theta.T)
    As = Avees.T.reshape(n_theta * NC, 1, NDIM + 1)
    zero_row = jnp.zeros((n_theta * NC, 1, NDIM + 1), dtype=theta.dtype)
    AsSquare = jnp.concatenate([As, zero_row], axis=1)
    Trels = _expm_taylor_2x2(AsSquare * (1.0 / NSTEP))
    return Trels[:, 0, 0], Trels[:, 0, 1]


def _sc_body(points_hbm, ctab_hbm, dtab_hbm, out_hbm,
             c0_v, c1_v, d0_v, d1_v, p0_v, p1_v,
             ob00, ob01, ob10, ob11, sin0, sin1, sout0, sout1):
    wid = lax.axis_index("s") * 2 + lax.axis_index("c")
    t0 = wid * T_PER_W
    pltpu.sync_copy(ctab_hbm.at[pl.ds(t0 * NC, NC)], c0_v)
    pltpu.sync_copy(ctab_hbm.at[pl.ds((t0 + 1) * NC, NC)], c1_v)
    pltpu.sync_copy(dtab_hbm.at[pl.ds(t0 * NC, NC)], d0_v)
    pltpu.sync_copy(dtab_hbm.at[pl.ds((t0 + 1) * NC, NC)], d1_v)
    inv_nc = 1.0 / NC
    pts = (p0_v, p1_v)
    obufs = ((ob00, ob01), (ob10, ob11))
    sins = (sin0, sin1)
    souts = (sout0, sout1)

    def in_copy(chunk, p):
        return pltpu.make_async_copy(
            points_hbm.at[pl.ds(chunk * CHUNK, CHUNK)], pts[p], sins[p])

    def out_copy(chunk, p, tt):
        return pltpu.make_async_copy(
            obufs[tt][p],
            out_hbm.at[pl.ds((t0 + tt) * NPOINTS + chunk * CHUNK, CHUNK)],
            souts[p])

    in_copy(0, 0).start()
    in_copy(1, 1).start()

    def chunk_body(j, carry):
        for p in range(2):
            chunk = 2 * j + p
            in_copy(chunk, p).wait()

            @pl.when(j > 0)
            def _():
                out_copy(chunk, p, 0).wait()
                out_copy(chunk, p, 1).wait()

            pv = pts[p]
            ob0 = obufs[0][p]
            ob1 = obufs[1][p]

            @plsc.parallel_loop(0, CHUNK // 16, 1, unroll=UNROLL)
            def _(i):
                base = i * 16
                x = pv[pl.ds(base, 16)]
                ii = jnp.clip(x, 0.0, float(NC - 1)).astype(jnp.int32)
                x0 = plsc.load_gather(c0_v, [ii]) * x + plsc.load_gather(d0_v, [ii])
                x1 = plsc.load_gather(c1_v, [ii]) * x + plsc.load_gather(d1_v, [ii])
                for _step in range(NSTEP - 1):
                    i0 = jnp.clip(x0, 0.0, float(NC - 1)).astype(jnp.int32)
                    i1 = jnp.clip(x1, 0.0, float(NC - 1)).astype(jnp.int32)
                    cv0 = plsc.load_gather(c0_v, [i0])
                    dv0 = plsc.load_gather(d0_v, [i0])
                    cv1 = plsc.load_gather(c1_v, [i1])
                    dv1 = plsc.load_gather(d1_v, [i1])
                    x0 = cv0 * x0 + dv0
                    x1 = cv1 * x1 + dv1
                ob0[pl.ds(base, 16)] = x0 * inv_nc
                ob1[pl.ds(base, 16)] = x1 * inv_nc

            out_copy(chunk, p, 0).start()
            out_copy(chunk, p, 1).start()

            @pl.when(j < NCHUNK // 2 - 1)
            def _():
                in_copy(chunk + 2, p).start()
        return carry

    lax.fori_loop(0, NCHUNK // 2, chunk_body, 0)
    for p in range(2):
        out_copy(NCHUNK - 2 + p, p, 0).wait()
        out_copy(NCHUNK - 2 + p, p, 1).wait()


@functools.partial(
    pl.kernel,
    out_type=jax.ShapeDtypeStruct((NTHETA * NPOINTS,), jnp.float32),
    mesh=plsc.VectorSubcoreMesh(core_axis_name="c", subcore_axis_name="s"),
    compiler_params=pltpu.CompilerParams(needs_layout_passes=False,
                                         skip_device_barrier=True),
    scratch_types=[
        pltpu.VMEM((NC,), jnp.float32),
        pltpu.VMEM((NC,), jnp.float32),
        pltpu.VMEM((NC,), jnp.float32),
        pltpu.VMEM((NC,), jnp.float32),
        pltpu.VMEM((CHUNK,), jnp.float32),
        pltpu.VMEM((CHUNK,), jnp.float32),
        pltpu.VMEM((CHUNK,), jnp.float32),
        pltpu.VMEM((CHUNK,), jnp.float32),
        pltpu.VMEM((CHUNK,), jnp.float32),
        pltpu.VMEM((CHUNK,), jnp.float32),
        pltpu.SemaphoreType.DMA,
        pltpu.SemaphoreType.DMA,
        pltpu.SemaphoreType.DMA,
        pltpu.SemaphoreType.DMA,
    ],
)
def _sc_transform(points_hbm, ctab_hbm, dtab_hbm, out_hbm,
                  c0_v, c1_v, d0_v, d1_v, p0_v, p1_v,
                  ob00, ob01, ob10, ob11, sin0, sin1, sout0, sout1):
    _sc_body(points_hbm, ctab_hbm, dtab_hbm, out_hbm,
             c0_v, c1_v, d0_v, d1_v, p0_v, p1_v,
             ob00, ob01, ob10, ob11, sin0, sin1, sout0, sout1)


def kernel(points, theta, basis):
    ctab, dtab = _build_tables(theta, basis)
    out = _sc_transform(points.reshape(NPOINTS) * float(NC),
                        ctab, dtab * float(NC))
    return out.reshape(NTHETA, NDIM, NPOINTS)
